# baseline (speedup 1.0000x reference)
import functools

import jax
import jax.numpy as jnp
from jax import lax
from jax.experimental import pallas as pl
from jax.experimental.pallas import tpu as pltpu
from jax.experimental.pallas import tpu_sc as plsc

EMB_DIM = 256
N_CODES = 8192
N_PAD = 8448
BM = 1024
CHUNK = 2816
SUB = 256
N_SUB = CHUNK // SUB


def _argmin_body(e2t_ref, x2t_ref, et2_ref, xt_ref, idx_ref, acc_val, acc_idx):
    c = pl.program_id(1)
    x2t = x2t_ref[...]
    big = jnp.iinfo(jnp.int32).max

    dists = []
    chunk_min = None
    for t in range(N_SUB):
        e_sub = et2_ref[pl.ds(t * SUB, SUB), :]
        sim2 = jnp.dot(e_sub, xt_ref[...], preferred_element_type=jnp.float32)
        d = (x2t + e2t_ref[pl.ds(t * SUB, SUB), :]) - sim2
        dists.append(d)
        m = jnp.min(d, axis=0, keepdims=True)
        chunk_min = m if chunk_min is None else jnp.minimum(chunk_min, m)

    chunk_idx = None
    for t in range(N_SUB):
        row = lax.broadcasted_iota(jnp.int32, (SUB, BM), 0) + (c * CHUNK + t * SUB)
        cand = jnp.min(jnp.where(dists[t] == chunk_min, row, big),
                       axis=0, keepdims=True)
        chunk_idx = cand if chunk_idx is None else jnp.minimum(chunk_idx, cand)

    @pl.when(c == 0)
    def _():
        acc_val[...] = chunk_min.astype(jnp.bfloat16).astype(jnp.float32)
        acc_idx[...] = chunk_idx

    @pl.when(c > 0)
    def _():
        better = chunk_min < acc_val[...]
        rounded = chunk_min.astype(jnp.bfloat16).astype(jnp.float32)
        acc_val[...] = jnp.where(better, rounded, acc_val[...])
        acc_idx[...] = jnp.where(better, chunk_idx, acc_idx[...])

    @pl.when(c == 2)
    def _():
        idx_ref[...] = acc_idx[...]


def _encode_indices(xt, et2, x2t, e2t):
    num_m = xt.shape[1] // BM
    return pl.pallas_call(
        _argmin_body,
        grid=(num_m, 3),
        in_specs=[
            pl.BlockSpec((CHUNK, 1), lambda i, c: (c, 0)),
            pl.BlockSpec((1, BM), lambda i, c: (0, i)),
            pl.BlockSpec((CHUNK, EMB_DIM), lambda i, c: (c, 0)),
            pl.BlockSpec((EMB_DIM, BM), lambda i, c: (0, i)),
        ],
        out_specs=pl.BlockSpec((1, BM), lambda i, c: (0, i)),
        out_shape=jax.ShapeDtypeStruct((1, xt.shape[1]), jnp.int32),
        scratch_shapes=[
            pltpu.VMEM((1, BM), jnp.float32),
            pltpu.VMEM((1, BM), jnp.int32),
        ],
    )(e2t, x2t, et2, xt)


NUM_TOKENS = 16384
NW = 32
ROWS_PER_W = NUM_TOKENS // NW
GCHUNK = 128


def _sc_gather(table, idx):
    mesh = plsc.VectorSubcoreMesh(core_axis_name="c", subcore_axis_name="s")

    @functools.partial(
        pl.kernel,
        mesh=mesh,
        out_type=jax.ShapeDtypeStruct((NUM_TOKENS, EMB_DIM), jnp.float32),
        scratch_types=[
            pltpu.VMEM((GCHUNK,), jnp.int32),
            pltpu.VMEM((GCHUNK, EMB_DIM), jnp.float32),
            pltpu.SemaphoreType.DMA,
        ],
    )
    def _gather_kernel(table_hbm, idx_hbm, out_hbm, idx_v, rows_v, sem):
        wid = lax.axis_index("s") * 2 + lax.axis_index("c")
        base = wid * ROWS_PER_W
        for cidx in range(ROWS_PER_W // GCHUNK):
            off = base + cidx * GCHUNK
            pltpu.sync_copy(idx_hbm.at[pl.ds(off, GCHUNK)], idx_v)
            pltpu.async_copy(table_hbm.at[idx_v], rows_v, sem).wait()
            pltpu.sync_copy(rows_v, out_hbm.at[pl.ds(off, GCHUNK)])

    return _gather_kernel(table, idx)


def kernel(x, embeddings):
    input_shape = x.shape
    flat = x.reshape(-1, EMB_DIM)
    x2 = jnp.sum(flat ** 2, axis=1, keepdims=True)
    e2 = jnp.sum(embeddings ** 2, axis=0)
    e2t = jnp.concatenate(
        [e2, jnp.full((N_PAD - N_CODES,), jnp.inf, jnp.float32)]).reshape(N_PAD, 1)
    et2 = jnp.concatenate(
        [(2.0 * embeddings).T,
         jnp.zeros((N_PAD - N_CODES, EMB_DIM), jnp.float32)], axis=0)
    xt = flat.T
    x2t = x2.reshape(1, -1)
    idx = _encode_indices(xt, et2, x2t, e2t)[0]
    quantized = (0.5 * _sc_gather(et2, idx)).reshape(input_shape)
    return x + lax.stop_gradient(quantized - x)


# speedup vs baseline: 1.0344x; 1.0344x over previous
import functools

import jax
import jax.numpy as jnp
from jax import lax
from jax.experimental import pallas as pl
from jax.experimental.pallas import tpu as pltpu
from jax.experimental.pallas import tpu_sc as plsc

EMB_DIM = 256
N_CODES = 8192
N_PAD = 8448
BM = 2048
CHUNK = 2816
SUB = 256
N_SUB = CHUNK // SUB


def _argmin_body(e2t_ref, x2t_ref, et2_ref, xt_ref, idx_ref, acc_val, acc_idx):
    c = pl.program_id(1)
    x2t = x2t_ref[...]
    big = jnp.iinfo(jnp.int32).max

    dists = []
    chunk_min = None
    for t in range(N_SUB):
        e_sub = et2_ref[pl.ds(t * SUB, SUB), :]
        sim2 = jnp.dot(e_sub, xt_ref[...], preferred_element_type=jnp.float32)
        d = (x2t + e2t_ref[pl.ds(t * SUB, SUB), :]) - sim2
        dists.append(d)
        m = jnp.min(d, axis=0, keepdims=True)
        chunk_min = m if chunk_min is None else jnp.minimum(chunk_min, m)

    chunk_idx = None
    for t in range(N_SUB):
        row = lax.broadcasted_iota(jnp.int32, (SUB, BM), 0) + (c * CHUNK + t * SUB)
        cand = jnp.min(jnp.where(dists[t] == chunk_min, row, big),
                       axis=0, keepdims=True)
        chunk_idx = cand if chunk_idx is None else jnp.minimum(chunk_idx, cand)

    @pl.when(c == 0)
    def _():
        acc_val[...] = chunk_min.astype(jnp.bfloat16).astype(jnp.float32)
        acc_idx[...] = chunk_idx

    @pl.when(c > 0)
    def _():
        better = chunk_min < acc_val[...]
        rounded = chunk_min.astype(jnp.bfloat16).astype(jnp.float32)
        acc_val[...] = jnp.where(better, rounded, acc_val[...])
        acc_idx[...] = jnp.where(better, chunk_idx, acc_idx[...])

    @pl.when(c == 2)
    def _():
        idx_ref[...] = acc_idx[...]


def _encode_indices(xt, et2, x2t, e2t):
    num_m = xt.shape[1] // BM
    return pl.pallas_call(
        _argmin_body,
        grid=(num_m, 3),
        in_specs=[
            pl.BlockSpec((CHUNK, 1), lambda i, c: (c, 0)),
            pl.BlockSpec((1, BM), lambda i, c: (0, i)),
            pl.BlockSpec((CHUNK, EMB_DIM), lambda i, c: (c, 0)),
            pl.BlockSpec((EMB_DIM, BM), lambda i, c: (0, i)),
        ],
        out_specs=pl.BlockSpec((1, BM), lambda i, c: (0, i)),
        out_shape=jax.ShapeDtypeStruct((1, xt.shape[1]), jnp.int32),
        scratch_shapes=[
            pltpu.VMEM((1, BM), jnp.float32),
            pltpu.VMEM((1, BM), jnp.int32),
        ],
    )(e2t, x2t, et2, xt)


NUM_TOKENS = 16384
NW = 32
ROWS_PER_W = NUM_TOKENS // NW
GCHUNK = 128


def _sc_gather(table, idx):
    mesh = plsc.VectorSubcoreMesh(core_axis_name="c", subcore_axis_name="s")

    @functools.partial(
        pl.kernel,
        mesh=mesh,
        out_type=jax.ShapeDtypeStruct((NUM_TOKENS, EMB_DIM), jnp.float32),
        scratch_types=[
            pltpu.VMEM((GCHUNK,), jnp.int32),
            pltpu.VMEM((GCHUNK, EMB_DIM), jnp.float32),
            pltpu.SemaphoreType.DMA,
        ],
    )
    def _gather_kernel(table_hbm, idx_hbm, out_hbm, idx_v, rows_v, sem):
        wid = lax.axis_index("s") * 2 + lax.axis_index("c")
        base = wid * ROWS_PER_W
        for cidx in range(ROWS_PER_W // GCHUNK):
            off = base + cidx * GCHUNK
            pltpu.sync_copy(idx_hbm.at[pl.ds(off, GCHUNK)], idx_v)
            pltpu.async_copy(table_hbm.at[idx_v], rows_v, sem).wait()
            pltpu.sync_copy(rows_v, out_hbm.at[pl.ds(off, GCHUNK)])

    return _gather_kernel(table, idx)


def kernel(x, embeddings):
    input_shape = x.shape
    flat = x.reshape(-1, EMB_DIM)
    x2 = jnp.sum(flat ** 2, axis=1, keepdims=True)
    e2 = jnp.sum(embeddings ** 2, axis=0)
    e2t = jnp.concatenate(
        [e2, jnp.full((N_PAD - N_CODES,), jnp.inf, jnp.float32)]).reshape(N_PAD, 1)
    et2 = jnp.concatenate(
        [(2.0 * embeddings).T,
         jnp.zeros((N_PAD - N_CODES, EMB_DIM), jnp.float32)], axis=0)
    xt = flat.T
    x2t = x2.reshape(1, -1)
    idx = _encode_indices(xt, et2, x2t, e2t)[0]
    quantized = _sc_gather(embeddings.T, idx).reshape(input_shape)
    return x + lax.stop_gradient(quantized - x)


# speedup vs baseline: 1.0431x; 1.0084x over previous
import functools

import jax
import jax.numpy as jnp
from jax import lax
from jax.experimental import pallas as pl
from jax.experimental.pallas import tpu as pltpu
from jax.experimental.pallas import tpu_sc as plsc

EMB_DIM = 256
N_CODES = 8192
N_PAD = 8448
BM = 2048
CHUNK = 2816
SUB = 704
N_SUB = CHUNK // SUB


def _argmin_body(e2t_ref, x2t_ref, et2_ref, xt_ref, idx_ref, acc_val, acc_idx):
    c = pl.program_id(1)
    x2t = x2t_ref[...]
    big = jnp.iinfo(jnp.int32).max

    dists = []
    chunk_min = None
    for t in range(N_SUB):
        e_sub = et2_ref[pl.ds(t * SUB, SUB), :]
        sim2 = jnp.dot(e_sub, xt_ref[...], preferred_element_type=jnp.float32)
        d = (x2t + e2t_ref[pl.ds(t * SUB, SUB), :]) - sim2
        dists.append(d)
        m = jnp.min(d, axis=0, keepdims=True)
        chunk_min = m if chunk_min is None else jnp.minimum(chunk_min, m)

    chunk_idx = None
    for t in range(N_SUB):
        row = lax.broadcasted_iota(jnp.int32, (SUB, BM), 0) + (c * CHUNK + t * SUB)
        cand = jnp.min(jnp.where(dists[t] == chunk_min, row, big),
                       axis=0, keepdims=True)
        chunk_idx = cand if chunk_idx is None else jnp.minimum(chunk_idx, cand)

    @pl.when(c == 0)
    def _():
        acc_val[...] = chunk_min.astype(jnp.bfloat16).astype(jnp.float32)
        acc_idx[...] = chunk_idx

    @pl.when(c > 0)
    def _():
        better = chunk_min < acc_val[...]
        rounded = chunk_min.astype(jnp.bfloat16).astype(jnp.float32)
        acc_val[...] = jnp.where(better, rounded, acc_val[...])
        acc_idx[...] = jnp.where(better, chunk_idx, acc_idx[...])

    @pl.when(c == 2)
    def _():
        idx_ref[...] = acc_idx[...]


def _encode_indices(xt, et2, x2t, e2t):
    num_m = xt.shape[1] // BM
    return pl.pallas_call(
        _argmin_body,
        grid=(num_m, 3),
        in_specs=[
            pl.BlockSpec((CHUNK, 1), lambda i, c: (c, 0)),
            pl.BlockSpec((1, BM), lambda i, c: (0, i)),
            pl.BlockSpec((CHUNK, EMB_DIM), lambda i, c: (c, 0)),
            pl.BlockSpec((EMB_DIM, BM), lambda i, c: (0, i)),
        ],
        out_specs=pl.BlockSpec((1, BM), lambda i, c: (0, i)),
        out_shape=jax.ShapeDtypeStruct((1, xt.shape[1]), jnp.int32),
        scratch_shapes=[
            pltpu.VMEM((1, BM), jnp.float32),
            pltpu.VMEM((1, BM), jnp.int32),
        ],
    )(e2t, x2t, et2, xt)


NUM_TOKENS = 16384
NW = 32
ROWS_PER_W = NUM_TOKENS // NW
GCHUNK = 128


def _sc_gather(table, idx):
    mesh = plsc.VectorSubcoreMesh(core_axis_name="c", subcore_axis_name="s")

    @functools.partial(
        pl.kernel,
        mesh=mesh,
        out_type=jax.ShapeDtypeStruct((NUM_TOKENS, EMB_DIM), jnp.float32),
        scratch_types=[
            pltpu.VMEM((GCHUNK,), jnp.int32),
            pltpu.VMEM((GCHUNK, EMB_DIM), jnp.float32),
            pltpu.SemaphoreType.DMA,
        ],
    )
    def _gather_kernel(table_hbm, idx_hbm, out_hbm, idx_v, rows_v, sem):
        wid = lax.axis_index("s") * 2 + lax.axis_index("c")
        base = wid * ROWS_PER_W
        for cidx in range(ROWS_PER_W // GCHUNK):
            off = base + cidx * GCHUNK
            pltpu.sync_copy(idx_hbm.at[pl.ds(off, GCHUNK)], idx_v)
            pltpu.async_copy(table_hbm.at[idx_v], rows_v, sem).wait()
            pltpu.sync_copy(rows_v, out_hbm.at[pl.ds(off, GCHUNK)])

    return _gather_kernel(table, idx)


def kernel(x, embeddings):
    input_shape = x.shape
    flat = x.reshape(-1, EMB_DIM)
    x2 = jnp.sum(flat ** 2, axis=1, keepdims=True)
    e2 = jnp.sum(embeddings ** 2, axis=0)
    e2t = jnp.concatenate(
        [e2, jnp.full((N_PAD - N_CODES,), jnp.inf, jnp.float32)]).reshape(N_PAD, 1)
    et2 = jnp.concatenate(
        [(2.0 * embeddings).T,
         jnp.zeros((N_PAD - N_CODES, EMB_DIM), jnp.float32)], axis=0)
    xt = flat.T
    x2t = x2.reshape(1, -1)
    idx = _encode_indices(xt, et2, x2t, e2t)[0]
    quantized = _sc_gather(embeddings.T, idx).reshape(input_shape)
    return x + lax.stop_gradient(quantized - x)


# speedup vs baseline: 1.0813x; 1.0366x over previous
import functools

import jax
import jax.numpy as jnp
from jax import lax
from jax.experimental import pallas as pl
from jax.experimental.pallas import tpu as pltpu
from jax.experimental.pallas import tpu_sc as plsc

EMB_DIM = 256
N_CODES = 8192
N_PAD = 8448
BM = 2048
CHUNK = 2816
SUB = 704
N_SUB = CHUNK // SUB


def _argmin_body(e2t_ref, x2t_ref, et2_ref, xt_ref, idx_ref, acc_val, acc_idx):
    c = pl.program_id(1)
    x2t = x2t_ref[...]
    big = jnp.iinfo(jnp.int32).max

    dists = []
    chunk_min = None
    for t in range(N_SUB):
        e_sub = et2_ref[pl.ds(t * SUB, SUB), :]
        sim2 = lax.dot_general(e_sub, xt_ref[...],
                               (((1,), (1,)), ((), ())),
                               preferred_element_type=jnp.float32)
        d = (x2t + e2t_ref[pl.ds(t * SUB, SUB), :]) - sim2
        dists.append(d)
        m = jnp.min(d, axis=0, keepdims=True)
        chunk_min = m if chunk_min is None else jnp.minimum(chunk_min, m)

    chunk_idx = None
    for t in range(N_SUB):
        row = lax.broadcasted_iota(jnp.int32, (SUB, BM), 0) + (c * CHUNK + t * SUB)
        cand = jnp.min(jnp.where(dists[t] == chunk_min, row, big),
                       axis=0, keepdims=True)
        chunk_idx = cand if chunk_idx is None else jnp.minimum(chunk_idx, cand)

    @pl.when(c == 0)
    def _():
        acc_val[...] = chunk_min.astype(jnp.bfloat16).astype(jnp.float32)
        acc_idx[...] = chunk_idx

    @pl.when(c > 0)
    def _():
        better = chunk_min < acc_val[...]
        rounded = chunk_min.astype(jnp.bfloat16).astype(jnp.float32)
        acc_val[...] = jnp.where(better, rounded, acc_val[...])
        acc_idx[...] = jnp.where(better, chunk_idx, acc_idx[...])

    @pl.when(c == 2)
    def _():
        idx_ref[...] = acc_idx[...]


def _encode_indices(xt, et2, x2t, e2t):
    num_m = xt.shape[0] // BM
    return pl.pallas_call(
        _argmin_body,
        grid=(num_m, 3),
        in_specs=[
            pl.BlockSpec((CHUNK, 1), lambda i, c: (c, 0)),
            pl.BlockSpec((1, BM), lambda i, c: (0, i)),
            pl.BlockSpec((CHUNK, EMB_DIM), lambda i, c: (c, 0)),
            pl.BlockSpec((BM, EMB_DIM), lambda i, c: (i, 0)),
        ],
        out_specs=pl.BlockSpec((1, BM), lambda i, c: (0, i)),
        out_shape=jax.ShapeDtypeStruct((1, xt.shape[0]), jnp.int32),
        scratch_shapes=[
            pltpu.VMEM((1, BM), jnp.float32),
            pltpu.VMEM((1, BM), jnp.int32),
        ],
    )(e2t, x2t, et2, xt)


NUM_TOKENS = 16384
NW = 32
ROWS_PER_W = NUM_TOKENS // NW
GCHUNK = 128


def _sc_gather(table, idx):
    mesh = plsc.VectorSubcoreMesh(core_axis_name="c", subcore_axis_name="s")

    @functools.partial(
        pl.kernel,
        mesh=mesh,
        out_type=jax.ShapeDtypeStruct((NUM_TOKENS, EMB_DIM), jnp.float32),
        scratch_types=[
            pltpu.VMEM((GCHUNK,), jnp.int32),
            pltpu.VMEM((GCHUNK, EMB_DIM), jnp.float32),
            pltpu.SemaphoreType.DMA,
        ],
    )
    def _gather_kernel(table_hbm, idx_hbm, out_hbm, idx_v, rows_v, sem):
        wid = lax.axis_index("s") * 2 + lax.axis_index("c")
        base = wid * ROWS_PER_W
        for cidx in range(ROWS_PER_W // GCHUNK):
            off = base + cidx * GCHUNK
            pltpu.sync_copy(idx_hbm.at[pl.ds(off, GCHUNK)], idx_v)
            pltpu.async_copy(table_hbm.at[idx_v], rows_v, sem).wait()
            pltpu.sync_copy(rows_v, out_hbm.at[pl.ds(off, GCHUNK)])

    return _gather_kernel(table, idx)


def kernel(x, embeddings):
    input_shape = x.shape
    flat = x.reshape(-1, EMB_DIM)
    x2 = jnp.sum(flat ** 2, axis=1, keepdims=True)
    e2 = jnp.sum(embeddings ** 2, axis=0)
    e2t = jnp.concatenate(
        [e2, jnp.full((N_PAD - N_CODES,), jnp.inf, jnp.float32)]).reshape(N_PAD, 1)
    et2 = jnp.concatenate(
        [(2.0 * embeddings).T,
         jnp.zeros((N_PAD - N_CODES, EMB_DIM), jnp.float32)], axis=0)
    x2t = x2.reshape(1, -1)
    idx = _encode_indices(flat, et2, x2t, e2t)[0]
    quantized = _sc_gather(embeddings.T, idx).reshape(input_shape)
    return x + lax.stop_gradient(quantized - x)
